# baseline (speedup 1.0000x reference)
import functools

import jax
import jax.numpy as jnp
from jax import lax
from jax.experimental import pallas as pl
from jax.experimental.pallas import tpu as pltpu
from jax.experimental.pallas import tpu_sc as plsc

N_NODES = 10000
N_EDGES = 320000
D = 128
D_E = 16
N_GRAPHS = 64
N_CLASSES = 16

NC, NS = 2, 16
NW = NC * NS
EPW = N_EDGES // NW
CH = 100
BCH = 25
NBLK = EPW // (CH * BCH)
CHA = 50
BCHA = 25
NBLKA = EPW // (CHA * BCHA)
SLOTS = 6
NPT = 624
TAIL_OFF = NS * NPT
TAIL = N_NODES - TAIL_OFF

_f32 = jnp.float32


def _spans(total, step=48):
    off = 0
    while off < total:
        yield off, min(step, total - off)
        off += step


def _zero_buf(buf, rows=CH):
    def zrow(i, c):
        for j in range(D // 16):
            buf[i, pl.ds(j * 16, 16)] = jnp.zeros((16,), _f32)
        return c
    lax.fori_loop(0, rows, zrow, 0)


def _zero_shared(nbase, sid, buf, dst, sem):
    hs = [pltpu.async_copy(buf.at[pl.ds(0, sz)],
                           dst.at[pl.ds(nbase + off, sz)], sem)
          for off, sz in _spans(NPT)]

    @pl.when(sid == NS - 1)
    def _():
        pltpu.sync_copy(buf.at[pl.ds(0, TAIL)], dst.at[pl.ds(TAIL_OFF, TAIL)])

    for h in hs:
        h.wait()


def _copy_out(nbase, sid, src, dst, bufs, sems):
    hs = [None, None]
    for i, (off, sz) in enumerate(_spans(NPT)):
        sl = i % 2
        if hs[sl] is not None:
            hs[sl].wait()
        pltpu.sync_copy(src.at[pl.ds(nbase + off, sz)],
                        bufs[sl].at[pl.ds(0, sz)])
        hs[sl] = pltpu.async_copy(bufs[sl].at[pl.ds(0, sz)],
                                  dst.at[pl.ds(nbase + off, sz)], sems[sl])
    for h in hs:
        if h is not None:
            h.wait()

    @pl.when(sid == NS - 1)
    def _():
        pltpu.sync_copy(src.at[pl.ds(TAIL_OFF, TAIL)],
                        bufs[0].at[pl.ds(0, TAIL)])
        pltpu.sync_copy(bufs[0].at[pl.ds(0, TAIL)],
                        dst.at[pl.ds(TAIL_OFF, TAIL)])


def _sc_aggr_body(rc_h, g_h, p0_out, p1_out, sh_aggr, ibuf,
                  *bufs_sems):
    gbufs = bufs_sems[:SLOTS]
    gsems = bufs_sems[SLOTS:2 * SLOTS]
    ssems = bufs_sems[2 * SLOTS:3 * SLOTS]
    cid = lax.axis_index("c")
    sid = lax.axis_index("s")
    wid = sid * NC + cid
    nbase = pl.multiple_of(sid * NPT, 8)

    _zero_buf(gbufs[0], rows=CHA)
    _zero_shared(nbase, sid, gbufs[0], sh_aggr, gsems[0])
    plsc.subcore_barrier()

    def blk(b, c):
        pltpu.sync_copy(rc_h.at[wid, b], ibuf)
        cps = [None] * BCHA
        scs = [None] * BCHA
        for k in range(BCHA):
            sl = k % SLOTS
            if k >= SLOTS:
                scs[k - SLOTS].wait()
            cps[k] = pltpu.async_copy(g_h.at[ibuf.at[BCHA + k]],
                                      gbufs[sl], gsems[sl])
            if k >= 1:
                cps[k - 1].wait()
                scs[k - 1] = pltpu.async_copy(
                    gbufs[(k - 1) % SLOTS], sh_aggr.at[ibuf.at[k - 1]],
                    ssems[(k - 1) % SLOTS], add=True)
        cps[BCHA - 1].wait()
        scs[BCHA - 1] = pltpu.async_copy(
            gbufs[(BCHA - 1) % SLOTS], sh_aggr.at[ibuf.at[BCHA - 1]],
            ssems[(BCHA - 1) % SLOTS], add=True)
        for k in range(max(0, BCHA - SLOTS), BCHA):
            scs[k].wait()
        return c
    lax.fori_loop(0, NBLKA, blk, 0)

    plsc.subcore_barrier()

    @pl.when(cid == 0)
    def _():
        _copy_out(nbase, sid, sh_aggr, p0_out, (gbufs[0], gbufs[1]),
                  (gsems[0], gsems[1]))

    @pl.when(cid == 1)
    def _():
        _copy_out(nbase, sid, sh_aggr, p1_out, (gbufs[0], gbufs[1]),
                  (gsems[0], gsems[1]))


def _sc_stats_body(row_h, attr_h, s0_out, s1_out, sh_stats, ibuf,
                   abuf, sb0, sb1, ss0, ss1):
    sbufs, ssems = (sb0, sb1), (ss0, ss1)
    cid = lax.axis_index("c")
    sid = lax.axis_index("s")
    wid = sid * NC + cid
    nbase = pl.multiple_of(sid * NPT, 8)

    _zero_buf(sb0)
    _zero_buf(sb1)
    _zero_shared(nbase, sid, sb0, sh_stats, ss0)

    one0 = jnp.where(lax.iota(jnp.int32, 16) == 0, 1.0, 0.0).astype(_f32)
    for sb in sbufs:
        def onerow(i, c, sb=sb):
            sb[i, pl.ds(D_E, 16)] = one0
            return c
        lax.fori_loop(0, CH, onerow, 0)

    plsc.subcore_barrier()

    def blk(b, c):
        pltpu.sync_copy(row_h.at[wid, b], ibuf)
        scs = [None] * BCH
        for k in range(BCH):
            sl = k % 2
            if k >= 2:
                scs[k - 2].wait()
            chunk_id = wid * (NBLK * BCH) + b * BCH + k
            pltpu.sync_copy(attr_h.at[chunk_id], abuf)
            sb = sbufs[sl]

            def build(i, c2, sb=sb):
                for r in range(10):
                    sb[i * 10 + r, pl.ds(0, D_E)] = \
                        abuf[i * 10 + r, pl.ds(0, D_E)]
                return c2
            lax.fori_loop(0, CH // 10, build, 0)
            scs[k] = pltpu.async_copy(sb, sh_stats.at[ibuf.at[k]],
                                      ssems[sl], add=True)
        scs[BCH - 2].wait()
        scs[BCH - 1].wait()
        return c
    lax.fori_loop(0, NBLK, blk, 0)

    plsc.subcore_barrier()

    @pl.when(cid == 0)
    def _():
        _copy_out(nbase, sid, sh_stats, s0_out, (sb0, sb1), (ss0, ss1))

    @pl.when(cid == 1)
    def _():
        _copy_out(nbase, sid, sh_stats, s1_out, (sb0, sb1), (ss0, ss1))


_SC_MESH = dict(core_axis_name="c", subcore_axis_name="s")


def _sc_aggr(rc, g):
    return pl.kernel(
        _sc_aggr_body,
        out_type=[jax.ShapeDtypeStruct((N_NODES, D), _f32)] * 2,
        mesh=plsc.VectorSubcoreMesh(**_SC_MESH),
        scratch_types=[pltpu.VMEM_SHARED((N_NODES, D), _f32),
                       pltpu.VMEM((2 * BCHA, CHA), jnp.int32)]
        + [pltpu.VMEM((CHA, D), _f32)] * SLOTS
        + [pltpu.SemaphoreType.DMA] * (2 * SLOTS),
    )(rc, g)


def _sc_stats(rowb, attr3):
    return pl.kernel(
        _sc_stats_body,
        out_type=[jax.ShapeDtypeStruct((N_NODES, D), _f32)] * 2,
        mesh=plsc.VectorSubcoreMesh(**_SC_MESH),
        scratch_types=[pltpu.VMEM_SHARED((N_NODES, D), _f32),
                       pltpu.VMEM((BCH, CH), jnp.int32),
                       pltpu.VMEM((CH, D_E), _f32),
                       pltpu.VMEM((CH, D), _f32),
                       pltpu.VMEM((CH, D), _f32)]
        + [pltpu.SemaphoreType.DMA] * 2,
    )(rowb, attr3)


_BN = 2000
_GRID = N_NODES // _BN


def _lin_body(x_ref, w_ref, b_ref, o_ref):
    o_ref[...] = (jnp.dot(x_ref[...], w_ref[...],
                          preferred_element_type=_f32) + b_ref[...])


def _tc_linear(x, w, b):
    return pl.pallas_call(
        _lin_body,
        grid=(_GRID,),
        in_specs=[pl.BlockSpec((_BN, x.shape[1]), lambda i: (i, 0)),
                  pl.BlockSpec(w.shape, lambda i: (0, 0)),
                  pl.BlockSpec(b.shape, lambda i: (0, 0))],
        out_specs=pl.BlockSpec((_BN, w.shape[1]), lambda i: (i, 0)),
        out_shape=jax.ShapeDtypeStruct((x.shape[0], w.shape[1]), _f32),
    )(x, w, b)


def _relu_aggr(p0, p1, s0, s1, we, be):
    s = s0 + s1
    agg = p0 + p1 + jnp.dot(s[:, 0:D_E], we, preferred_element_type=_f32)
    agg = agg + s[:, D_E:D_E + 1] * be
    return jnp.maximum(agg, 0.0)


def _mid_body(p0, p1, s0, s1, we, be, w2, b2, o_ref):
    h = _relu_aggr(p0[...], p1[...], s0[...], s1[...], we[...], be[...])
    o_ref[...] = jnp.dot(h, w2[...], preferred_element_type=_f32) + b2[...]


def _tc_mid(p0, p1, s0, s1, we, be, w2, b2):
    big = pl.BlockSpec((_BN, D), lambda i: (i, 0))
    full = lambda a: pl.BlockSpec(a.shape, lambda i: (0, 0))
    return pl.pallas_call(
        _mid_body,
        grid=(_GRID,),
        in_specs=[big, big, big, big, full(we), full(be), full(w2), full(b2)],
        out_specs=big,
        out_shape=jax.ShapeDtypeStruct((N_NODES, D), _f32),
    )(p0, p1, s0, s1, we, be, w2, b2)


def _final_body(p0, p1, s0, s1, we, be, batch_ref, wc, bc, o_ref,
                acc_s, acc_n):
    i = pl.program_id(0)

    @pl.when(i == 0)
    def _():
        acc_s[...] = jnp.zeros_like(acc_s)
        acc_n[...] = jnp.zeros_like(acc_n)

    h = _relu_aggr(p0[...], p1[...], s0[...], s1[...], we[...], be[...])
    b = batch_ref[...]
    onehot = (lax.broadcasted_iota(jnp.int32, (_BN, N_GRAPHS), 1)
              == b).astype(_f32)
    dn = (((0,), (0,)), ((), ()))
    acc_s[...] += lax.dot_general(onehot, h, dn, preferred_element_type=_f32)
    acc_n[...] += lax.dot_general(onehot, jnp.ones_like(h), dn,
                                  preferred_element_type=_f32)

    @pl.when(i == _GRID - 1)
    def _():
        pooled = acc_s[...] / acc_n[...]
        o_ref[...] = (jnp.dot(pooled, wc[...], preferred_element_type=_f32)
                      + bc[...])


def _tc_final(p0, p1, s0, s1, we, be, batch2d, wc, bc):
    big = pl.BlockSpec((_BN, D), lambda i: (i, 0))
    full = lambda a: pl.BlockSpec(a.shape, lambda i: (0, 0))
    return pl.pallas_call(
        _final_body,
        grid=(_GRID,),
        in_specs=[big, big, big, big, full(we), full(be),
                  pl.BlockSpec((_BN, 1), lambda i: (i, 0)),
                  full(wc), full(bc)],
        out_specs=pl.BlockSpec((N_GRAPHS, N_CLASSES), lambda i: (0, 0)),
        out_shape=jax.ShapeDtypeStruct((N_GRAPHS, N_CLASSES), _f32),
        scratch_shapes=[pltpu.VMEM((N_GRAPHS, D), _f32),
                        pltpu.VMEM((N_GRAPHS, D), _f32)],
    )(p0, p1, s0, s1, we, be, batch2d, wc, bc)


def kernel(x, edge_index, edge_attr, batch,
           W1, b1, We1, be1, W2, b2, We2, be2, Wc, bc):
    row = edge_index[0].astype(jnp.int32).reshape(NW, NBLK, BCH, CH)
    col = edge_index[1].astype(jnp.int32).reshape(NW, NBLK, BCH, CH)
    rowa = edge_index[0].astype(jnp.int32).reshape(NW, NBLKA, BCHA, CHA)
    cola = edge_index[1].astype(jnp.int32).reshape(NW, NBLKA, BCHA, CHA)
    rc = jnp.concatenate([rowa, cola], axis=2)
    attr3 = edge_attr.reshape(N_EDGES // CH, CH, D_E)
    batch2d = batch.astype(jnp.int32).reshape(N_NODES, 1)
    b1r, be1r = b1.reshape(1, D), be1.reshape(1, D)
    b2r, be2r = b2.reshape(1, D), be2.reshape(1, D)
    bcr = bc.reshape(1, N_CLASSES)

    s0, s1 = _sc_stats(row, attr3)
    g1 = _tc_linear(x, W1, b1r)
    p0, p1 = _sc_aggr(rc, g1)
    g2 = _tc_mid(p0, p1, s0, s1, We1, be1r, W2, b2r)
    q0, q1 = _sc_aggr(rc, g2)
    return _tc_final(q0, q1, s0, s1, We2, be2r, batch2d, Wc, bcr)


# speedup vs baseline: 1.0902x; 1.0902x over previous
import functools

import jax
import jax.numpy as jnp
from jax import lax
from jax.experimental import pallas as pl
from jax.experimental.pallas import tpu as pltpu
from jax.experimental.pallas import tpu_sc as plsc

N_NODES = 10000
N_EDGES = 320000
D = 128
D_E = 16
N_GRAPHS = 64
N_CLASSES = 16

NC, NS = 2, 16
NW = NC * NS
EPW = N_EDGES // NW
CH = 100
BCH = 25
NBLK = EPW // (CH * BCH)
NPT = 624
TAIL_OFF = NS * NPT
TAIL = N_NODES - TAIL_OFF

_f32 = jnp.float32


def _spans(total, step=96):
    off = 0
    while off < total:
        yield off, min(step, total - off)
        off += step


def _zero_buf(buf):
    def zrow(i, c):
        for j in range(D // 16):
            buf[i, pl.ds(j * 16, 16)] = jnp.zeros((16,), _f32)
        return c
    lax.fori_loop(0, CH, zrow, 0)


def _zero_shared(nbase, sid, buf, dst, sem):
    hs = [pltpu.async_copy(buf.at[pl.ds(0, sz)],
                           dst.at[pl.ds(nbase + off, sz)], sem)
          for off, sz in _spans(NPT)]

    @pl.when(sid == NS - 1)
    def _():
        pltpu.sync_copy(buf.at[pl.ds(0, TAIL)], dst.at[pl.ds(TAIL_OFF, TAIL)])

    for h in hs:
        h.wait()


def _copy_out(nbase, sid, src, dst, bufs, sems):
    hs = [None, None]
    for i, (off, sz) in enumerate(_spans(NPT)):
        sl = i % 2
        if hs[sl] is not None:
            hs[sl].wait()
        pltpu.sync_copy(src.at[pl.ds(nbase + off, sz)],
                        bufs[sl].at[pl.ds(0, sz)])
        hs[sl] = pltpu.async_copy(bufs[sl].at[pl.ds(0, sz)],
                                  dst.at[pl.ds(nbase + off, sz)], sems[sl])
    for h in hs:
        if h is not None:
            h.wait()

    @pl.when(sid == NS - 1)
    def _():
        pltpu.sync_copy(src.at[pl.ds(TAIL_OFF, TAIL)],
                        bufs[0].at[pl.ds(0, TAIL)])
        pltpu.sync_copy(bufs[0].at[pl.ds(0, TAIL)],
                        dst.at[pl.ds(TAIL_OFF, TAIL)])


def _sc_aggr_body(rc_h, g_h, p0_out, p1_out, sh_aggr, ibuf,
                  gb0, gb1, gb2, gs0, gs1, gs2, ss0, ss1, ss2):
    gbufs, gsems, ssems = (gb0, gb1, gb2), (gs0, gs1, gs2), (ss0, ss1, ss2)
    cid = lax.axis_index("c")
    sid = lax.axis_index("s")
    wid = sid * NC + cid
    nbase = pl.multiple_of(sid * NPT, 8)

    _zero_buf(gb0)
    _zero_shared(nbase, sid, gb0, sh_aggr, gs0)
    plsc.subcore_barrier()

    def blk(b, c):
        pltpu.sync_copy(rc_h.at[wid, b], ibuf)
        cps = [None] * BCH
        scs = [None] * BCH
        for k in range(BCH):
            sl = k % 3
            if k >= 3:
                scs[k - 3].wait()
            cps[k] = pltpu.async_copy(g_h.at[ibuf.at[BCH + k]],
                                      gbufs[sl], gsems[sl])
            if k >= 1:
                cps[k - 1].wait()
                scs[k - 1] = pltpu.async_copy(
                    gbufs[(k - 1) % 3], sh_aggr.at[ibuf.at[k - 1]],
                    ssems[(k - 1) % 3], add=True)
        cps[BCH - 1].wait()
        scs[BCH - 1] = pltpu.async_copy(
            gbufs[(BCH - 1) % 3], sh_aggr.at[ibuf.at[BCH - 1]],
            ssems[(BCH - 1) % 3], add=True)
        for k in (BCH - 3, BCH - 2, BCH - 1):
            scs[k].wait()
        return c
    lax.fori_loop(0, NBLK, blk, 0)

    plsc.subcore_barrier()

    @pl.when(cid == 0)
    def _():
        _copy_out(nbase, sid, sh_aggr, p0_out, (gb0, gb1), (gs0, gs1))

    @pl.when(cid == 1)
    def _():
        _copy_out(nbase, sid, sh_aggr, p1_out, (gb0, gb1), (gs0, gs1))


def _sc_stats_body(row_h, attr_h, s0_out, s1_out, sh_stats, ibuf,
                   abuf, sb0, sb1, ss0, ss1):
    sbufs, ssems = (sb0, sb1), (ss0, ss1)
    cid = lax.axis_index("c")
    sid = lax.axis_index("s")
    wid = sid * NC + cid
    nbase = pl.multiple_of(sid * NPT, 8)

    _zero_buf(sb0)
    _zero_buf(sb1)
    _zero_shared(nbase, sid, sb0, sh_stats, ss0)

    one0 = jnp.where(lax.iota(jnp.int32, 16) == 0, 1.0, 0.0).astype(_f32)
    for sb in sbufs:
        def onerow(i, c, sb=sb):
            sb[i, pl.ds(D_E, 16)] = one0
            return c
        lax.fori_loop(0, CH, onerow, 0)

    plsc.subcore_barrier()

    def blk(b, c):
        pltpu.sync_copy(row_h.at[wid, b], ibuf)
        scs = [None] * BCH
        for k in range(BCH):
            sl = k % 2
            if k >= 2:
                scs[k - 2].wait()
            chunk_id = wid * (NBLK * BCH) + b * BCH + k
            pltpu.sync_copy(attr_h.at[chunk_id], abuf)
            sb = sbufs[sl]

            def build(i, c2, sb=sb):
                for r in range(10):
                    sb[i * 10 + r, pl.ds(0, D_E)] = \
                        abuf[i * 10 + r, pl.ds(0, D_E)]
                return c2
            lax.fori_loop(0, CH // 10, build, 0)
            scs[k] = pltpu.async_copy(sb, sh_stats.at[ibuf.at[k]],
                                      ssems[sl], add=True)
        scs[BCH - 2].wait()
        scs[BCH - 1].wait()
        return c
    lax.fori_loop(0, NBLK, blk, 0)

    plsc.subcore_barrier()

    @pl.when(cid == 0)
    def _():
        _copy_out(nbase, sid, sh_stats, s0_out, (sb0, sb1), (ss0, ss1))

    @pl.when(cid == 1)
    def _():
        _copy_out(nbase, sid, sh_stats, s1_out, (sb0, sb1), (ss0, ss1))


_SC_MESH = dict(core_axis_name="c", subcore_axis_name="s")


def _sc_aggr(rc, g):
    return pl.kernel(
        _sc_aggr_body,
        out_type=[jax.ShapeDtypeStruct((N_NODES, D), _f32)] * 2,
        mesh=plsc.VectorSubcoreMesh(**_SC_MESH),
        scratch_types=[pltpu.VMEM_SHARED((N_NODES, D), _f32),
                       pltpu.VMEM((2 * BCH, CH), jnp.int32),
                       pltpu.VMEM((CH, D), _f32),
                       pltpu.VMEM((CH, D), _f32),
                       pltpu.VMEM((CH, D), _f32)]
        + [pltpu.SemaphoreType.DMA] * 6,
    )(rc, g)


def _sc_stats(rowb, attr3):
    return pl.kernel(
        _sc_stats_body,
        out_type=[jax.ShapeDtypeStruct((N_NODES, D), _f32)] * 2,
        mesh=plsc.VectorSubcoreMesh(**_SC_MESH),
        scratch_types=[pltpu.VMEM_SHARED((N_NODES, D), _f32),
                       pltpu.VMEM((BCH, CH), jnp.int32),
                       pltpu.VMEM((CH, D_E), _f32),
                       pltpu.VMEM((CH, D), _f32),
                       pltpu.VMEM((CH, D), _f32)]
        + [pltpu.SemaphoreType.DMA] * 2,
    )(rowb, attr3)


_BN = 2000
_GRID = N_NODES // _BN


def _lin_body(x_ref, w_ref, b_ref, o_ref):
    o_ref[...] = (jnp.dot(x_ref[...], w_ref[...],
                          preferred_element_type=_f32) + b_ref[...])


def _tc_linear(x, w, b):
    return pl.pallas_call(
        _lin_body,
        grid=(_GRID,),
        in_specs=[pl.BlockSpec((_BN, x.shape[1]), lambda i: (i, 0)),
                  pl.BlockSpec(w.shape, lambda i: (0, 0)),
                  pl.BlockSpec(b.shape, lambda i: (0, 0))],
        out_specs=pl.BlockSpec((_BN, w.shape[1]), lambda i: (i, 0)),
        out_shape=jax.ShapeDtypeStruct((x.shape[0], w.shape[1]), _f32),
    )(x, w, b)


def _relu_aggr(p0, p1, s0, s1, we, be):
    s = s0 + s1
    agg = p0 + p1 + jnp.dot(s[:, 0:D_E], we, preferred_element_type=_f32)
    agg = agg + s[:, D_E:D_E + 1] * be
    return jnp.maximum(agg, 0.0)


def _mid_body(p0, p1, s0, s1, we, be, w2, b2, o_ref):
    h = _relu_aggr(p0[...], p1[...], s0[...], s1[...], we[...], be[...])
    o_ref[...] = jnp.dot(h, w2[...], preferred_element_type=_f32) + b2[...]


def _tc_mid(p0, p1, s0, s1, we, be, w2, b2):
    big = pl.BlockSpec((_BN, D), lambda i: (i, 0))
    full = lambda a: pl.BlockSpec(a.shape, lambda i: (0, 0))
    return pl.pallas_call(
        _mid_body,
        grid=(_GRID,),
        in_specs=[big, big, big, big, full(we), full(be), full(w2), full(b2)],
        out_specs=big,
        out_shape=jax.ShapeDtypeStruct((N_NODES, D), _f32),
    )(p0, p1, s0, s1, we, be, w2, b2)


def _final_body(p0, p1, s0, s1, we, be, batch_ref, wc, bc, o_ref,
                acc_s, acc_n):
    i = pl.program_id(0)

    @pl.when(i == 0)
    def _():
        acc_s[...] = jnp.zeros_like(acc_s)
        acc_n[...] = jnp.zeros_like(acc_n)

    h = _relu_aggr(p0[...], p1[...], s0[...], s1[...], we[...], be[...])
    b = batch_ref[...]
    onehot = (lax.broadcasted_iota(jnp.int32, (_BN, N_GRAPHS), 1)
              == b).astype(_f32)
    dn = (((0,), (0,)), ((), ()))
    acc_s[...] += lax.dot_general(onehot, h, dn, preferred_element_type=_f32)
    acc_n[...] += lax.dot_general(onehot, jnp.ones_like(h), dn,
                                  preferred_element_type=_f32)

    @pl.when(i == _GRID - 1)
    def _():
        pooled = acc_s[...] / acc_n[...]
        o_ref[...] = (jnp.dot(pooled, wc[...], preferred_element_type=_f32)
                      + bc[...])


def _tc_final(p0, p1, s0, s1, we, be, batch2d, wc, bc):
    big = pl.BlockSpec((_BN, D), lambda i: (i, 0))
    full = lambda a: pl.BlockSpec(a.shape, lambda i: (0, 0))
    return pl.pallas_call(
        _final_body,
        grid=(_GRID,),
        in_specs=[big, big, big, big, full(we), full(be),
                  pl.BlockSpec((_BN, 1), lambda i: (i, 0)),
                  full(wc), full(bc)],
        out_specs=pl.BlockSpec((N_GRAPHS, N_CLASSES), lambda i: (0, 0)),
        out_shape=jax.ShapeDtypeStruct((N_GRAPHS, N_CLASSES), _f32),
        scratch_shapes=[pltpu.VMEM((N_GRAPHS, D), _f32),
                        pltpu.VMEM((N_GRAPHS, D), _f32)],
    )(p0, p1, s0, s1, we, be, batch2d, wc, bc)


def kernel(x, edge_index, edge_attr, batch,
           W1, b1, We1, be1, W2, b2, We2, be2, Wc, bc):
    row = edge_index[0].astype(jnp.int32).reshape(NW, NBLK, BCH, CH)
    col = edge_index[1].astype(jnp.int32).reshape(NW, NBLK, BCH, CH)
    rc = jnp.concatenate([row, col], axis=2)
    attr3 = edge_attr.reshape(N_EDGES // CH, CH, D_E)
    batch2d = batch.astype(jnp.int32).reshape(N_NODES, 1)
    b1r, be1r = b1.reshape(1, D), be1.reshape(1, D)
    b2r, be2r = b2.reshape(1, D), be2.reshape(1, D)
    bcr = bc.reshape(1, N_CLASSES)

    s0, s1 = _sc_stats(row, attr3)
    g1 = _tc_linear(x, W1, b1r)
    p0, p1 = _sc_aggr(rc, g1)
    g2 = _tc_mid(p0, p1, s0, s1, We1, be1r, W2, b2r)
    q0, q1 = _sc_aggr(rc, g2)
    return _tc_final(q0, q1, s0, s1, We2, be2r, batch2d, Wc, bcr)
